# Initial kernel scaffold; baseline (speedup 1.0000x reference)
import jax
import jax.numpy as jnp
from jax.experimental import pallas as pl


def kernel(x, horizontal_lines, vertical_lines):
    raise NotImplementedError("write your pallas kernel here")


# speedup vs baseline: 4.7166x; 4.7166x over previous
import jax
import jax.numpy as jnp
from jax.experimental import pallas as pl
from jax.experimental.pallas import tpu as pltpu

GRID = 32


def _label_kernel(h_ref, v_ref, out_ref):
    height, width = out_ref.shape

    def axis_ranks(lines_ref, pos):
        best_d = jnp.full(pos.shape, jnp.inf, dtype=jnp.float32)
        best_l = jnp.zeros(pos.shape, dtype=jnp.float32)
        for j in range(GRID):
            lj = lines_ref[j]
            d = jnp.abs(pos - lj)
            take = (d < best_d) | ((d == best_d) & (lj < best_l))
            best_d = jnp.where(take, d, best_d)
            best_l = jnp.where(take, lj, best_l)
        rank = jnp.zeros(pos.shape, dtype=jnp.int32)
        for j in range(GRID):
            lj = lines_ref[j]
            rank = rank + jnp.where(lj < best_l, 1, 0).astype(jnp.int32)
        return rank

    yy = jax.lax.broadcasted_iota(jnp.int32, (height, 1), 0).astype(jnp.float32)
    xx = jax.lax.broadcasted_iota(jnp.int32, (1, width), 1).astype(jnp.float32)
    nh = axis_ranks(h_ref, yy)
    nv = axis_ranks(v_ref, xx)
    out_ref[...] = nh * GRID + nv


def kernel(x, horizontal_lines, vertical_lines):
    _, _, height, width = x.shape
    return pl.pallas_call(
        _label_kernel,
        out_shape=jax.ShapeDtypeStruct((height, width), jnp.int32),
        in_specs=[
            pl.BlockSpec(memory_space=pltpu.SMEM),
            pl.BlockSpec(memory_space=pltpu.SMEM),
        ],
    )(horizontal_lines, vertical_lines)
